# Initial kernel scaffold; baseline (speedup 1.0000x reference)
import jax
import jax.numpy as jnp
from jax.experimental import pallas as pl


def kernel(hidden_states, gate_w, gate_b, up_w, up_b, gp_w, gp_b, down_w, down_b):
    raise NotImplementedError("write your pallas kernel here")


# speedup vs baseline: 1.5086x; 1.5086x over previous
import functools

import jax
import jax.numpy as jnp
from jax import lax
from jax.experimental import pallas as pl
from jax.experimental.pallas import tpu as pltpu
from jax.experimental.pallas import tpu_sc as plsc

E = 64
CAP = 128
H = 1024
F = 512
T = 2048
DUMMY = E * CAP
BUF_ROWS = E * CAP + CAP

NC = 2
NS = 16
NW = NC * NS
L = 16
TPW = T // NW


def _router_body(x_ref, gw_ref, gb_ref, slotd_ref, slotc_ref, wk_ref):
    x = x_ref[...]
    gw = gw_ref[...]
    logits = jnp.dot(x, gw, preferred_element_type=jnp.float32) + gb_ref[...]
    lmax = jnp.max(logits, axis=1, keepdims=True)
    sumexp = jnp.sum(jnp.exp(logits - lmax), axis=1, keepdims=True)
    p = 1.0 / sumexp
    ids = lax.broadcasted_iota(jnp.int32, (T, E), 1)
    eid = jnp.min(jnp.where(logits == lmax, ids, E), axis=1, keepdims=True)
    oh = (ids == eid).astype(jnp.int32)
    cs = oh
    shift = 1
    while shift < T:
        cs = cs + jnp.concatenate(
            [jnp.zeros((shift, E), jnp.int32), cs[: T - shift]], axis=0)
        shift *= 2
    pos = jnp.sum(cs * oh, axis=1, keepdims=True) - 1
    keep = pos < CAP
    slotd_ref[...] = jnp.where(keep, eid * CAP + pos, DUMMY)
    slotc_ref[...] = eid * CAP + jnp.minimum(pos, CAP - 1)
    wk_ref[...] = jnp.where(keep, p, 0.0)


_router_call = pl.pallas_call(
    _router_body,
    out_shape=(
        jax.ShapeDtypeStruct((T, 1), jnp.int32),
        jax.ShapeDtypeStruct((T, 1), jnp.int32),
        jax.ShapeDtypeStruct((T, 1), jnp.float32),
    ),
)


def _dispatch_body(x_hbm, slot_hbm, buf_hbm, idx_v, rows_v, sem):
    wid = lax.axis_index("s") * NC + lax.axis_index("c")
    base = wid * TPW
    pltpu.sync_copy(slot_hbm.at[pl.ds(base, TPW)], idx_v)
    pltpu.sync_copy(x_hbm.at[pl.ds(base, TPW)], rows_v)
    pltpu.async_copy(rows_v, buf_hbm.at[idx_v], sem).wait()


@functools.cache
def _sc_kernels():
    mesh = plsc.VectorSubcoreMesh(
        core_axis_name="c", subcore_axis_name="s", num_cores=NC, num_subcores=NS)
    dispatch = pl.kernel(
        _dispatch_body,
        out_type=jax.ShapeDtypeStruct((BUF_ROWS, H), jnp.float32),
        mesh=mesh,
        scratch_types=[
            pltpu.VMEM((TPW,), jnp.int32),
            pltpu.VMEM((TPW, H), jnp.float32),
            pltpu.SemaphoreType.DMA,
        ],
    )
    combine = pl.kernel(
        _combine_body,
        out_type=jax.ShapeDtypeStruct((T, H), jnp.float32),
        mesh=mesh,
        scratch_types=[
            pltpu.VMEM((TPW,), jnp.int32),
            pltpu.VMEM((TPW, H), jnp.float32),
            pltpu.VMEM((TPW,), jnp.float32),
            pltpu.SemaphoreType.DMA,
        ],
    )
    return dispatch, combine


def _ffn_body(xb_ref, gpw_ref, gpb_ref, upw_ref, upb_ref, dw_ref, db_ref, out_ref):
    xb = xb_ref[...]
    g = jnp.dot(xb, gpw_ref[0], preferred_element_type=jnp.float32) + gpb_ref[0]
    u = jnp.dot(xb, upw_ref[0], preferred_element_type=jnp.float32) + upb_ref[0]
    inter = g * (1.0 / (1.0 + jnp.exp(-g))) * u
    out_ref[...] = (
        jnp.dot(inter, dw_ref[0], preferred_element_type=jnp.float32) + db_ref[0])


_ffn_call = pl.pallas_call(
    _ffn_body,
    grid=(E,),
    in_specs=[
        pl.BlockSpec((CAP, H), lambda e: (e, 0)),
        pl.BlockSpec((1, H, F), lambda e: (e, 0, 0)),
        pl.BlockSpec((1, 1, F), lambda e: (e, 0, 0)),
        pl.BlockSpec((1, H, F), lambda e: (e, 0, 0)),
        pl.BlockSpec((1, 1, F), lambda e: (e, 0, 0)),
        pl.BlockSpec((1, F, H), lambda e: (e, 0, 0)),
        pl.BlockSpec((1, 1, H), lambda e: (e, 0, 0)),
    ],
    out_specs=pl.BlockSpec((CAP, H), lambda e: (e, 0)),
    out_shape=jax.ShapeDtypeStruct((BUF_ROWS, H), jnp.float32),
    compiler_params=pltpu.CompilerParams(
        dimension_semantics=("arbitrary",)),
)


def _combine_body(oute_hbm, slot_hbm, wk_hbm, final_hbm, idx_v, rows_v, w_v, sem):
    wid = lax.axis_index("s") * NC + lax.axis_index("c")
    base = wid * TPW
    pltpu.sync_copy(slot_hbm.at[pl.ds(base, TPW)], idx_v)
    pltpu.sync_copy(wk_hbm.at[pl.ds(base, TPW)], w_v)
    pltpu.async_copy(oute_hbm.at[idx_v], rows_v, sem).wait()

    def group_fn(g, carry):
        wreg = w_v[pl.ds(g * L, L)]
        for k in range(L):
            w = wreg[k]
            i = g * L + k

            def chunk_fn(j, c, i=i, w=w):
                sl = pl.ds(j * L, L)
                rows_v[i, sl] = rows_v[i, sl] * w
                return c

            carry = lax.fori_loop(0, H // L, chunk_fn, carry)
        return carry

    lax.fori_loop(0, TPW // L, group_fn, 0)
    pltpu.sync_copy(rows_v, final_hbm.at[pl.ds(base, TPW)])


def kernel(hidden_states, gate_w, gate_b, up_w, up_b, gp_w, gp_b, down_w, down_b):
    b, s, h = hidden_states.shape
    x = hidden_states.reshape(T, H)
    dispatch, combine = _sc_kernels()
    slotd, slotc, wk = _router_call(x, gate_w, gate_b.reshape(1, E))
    buf = dispatch(x, slotd.reshape(T))
    oute = _ffn_call(
        buf,
        gp_w, gp_b.reshape(E, 1, F),
        up_w, up_b.reshape(E, 1, F),
        down_w, down_b.reshape(E, 1, H),
    )
    final = combine(oute, slotc.reshape(T), wk.reshape(T))
    return final.reshape(b, s, h)


# speedup vs baseline: 1.6321x; 1.0819x over previous
import functools

import jax
import jax.numpy as jnp
from jax import lax
from jax.experimental import pallas as pl
from jax.experimental.pallas import tpu as pltpu
from jax.experimental.pallas import tpu_sc as plsc

E = 64
CAP = 128
H = 1024
F = 512
T = 2048
DUMMY = E * CAP
BUF_ROWS = E * CAP + CAP

NC = 2
NS = 16
NW = NC * NS
L = 16
TPW = T // NW


def _router_body(x_ref, gw_ref, gb_ref, slotd_ref, slotc_ref, wk_ref):
    x = x_ref[...]
    gw = gw_ref[...]
    logits = jnp.dot(x, gw, preferred_element_type=jnp.float32) + gb_ref[...]
    lmax = jnp.max(logits, axis=1, keepdims=True)
    sumexp = jnp.sum(jnp.exp(logits - lmax), axis=1, keepdims=True)
    p = 1.0 / sumexp
    ids = lax.broadcasted_iota(jnp.int32, (T, E), 1)
    eid = jnp.min(jnp.where(logits == lmax, ids, E), axis=1, keepdims=True)
    oh = (ids == eid).astype(jnp.int32)
    cs = oh
    shift = 1
    while shift < T:
        cs = cs + jnp.concatenate(
            [jnp.zeros((shift, E), jnp.int32), cs[: T - shift]], axis=0)
        shift *= 2
    pos = jnp.sum(cs * oh, axis=1, keepdims=True) - 1
    keep = pos < CAP
    slotd_ref[...] = jnp.where(keep, eid * CAP + pos, DUMMY)
    slotc_ref[...] = eid * CAP + jnp.minimum(pos, CAP - 1)
    wk_ref[...] = jnp.where(keep, p, 0.0)


_router_call = pl.pallas_call(
    _router_body,
    out_shape=(
        jax.ShapeDtypeStruct((T, 1), jnp.int32),
        jax.ShapeDtypeStruct((T, 1), jnp.int32),
        jax.ShapeDtypeStruct((T, 1), jnp.float32),
    ),
)


def _dispatch_body(x_hbm, slot_hbm, buf_hbm, idx_v, rows_v, sem):
    wid = lax.axis_index("s") * NC + lax.axis_index("c")
    base = wid * TPW
    pltpu.sync_copy(slot_hbm.at[pl.ds(base, TPW)], idx_v)
    pltpu.sync_copy(x_hbm.at[pl.ds(base, TPW)], rows_v)
    pltpu.async_copy(rows_v, buf_hbm.at[idx_v], sem).wait()


@functools.cache
def _sc_kernels():
    mesh = plsc.VectorSubcoreMesh(
        core_axis_name="c", subcore_axis_name="s", num_cores=NC, num_subcores=NS)
    dispatch = pl.kernel(
        _dispatch_body,
        out_type=jax.ShapeDtypeStruct((BUF_ROWS, H), jnp.float32),
        mesh=mesh,
        scratch_types=[
            pltpu.VMEM((TPW,), jnp.int32),
            pltpu.VMEM((TPW, H), jnp.float32),
            pltpu.SemaphoreType.DMA,
        ],
    )
    combine = pl.kernel(
        _combine_body,
        out_type=jax.ShapeDtypeStruct((T, H), jnp.float32),
        mesh=mesh,
        scratch_types=[
            pltpu.VMEM((TPW,), jnp.int32),
            pltpu.VMEM((TPW, H), jnp.float32),
            pltpu.VMEM((TPW,), jnp.float32),
            pltpu.SemaphoreType.DMA,
        ],
    )
    return dispatch, combine


def _ffn_body(xb_ref, gpw_ref, gpb_ref, upw_ref, upb_ref, dw_ref, db_ref, out_ref):
    xb = xb_ref[...]
    g = jnp.dot(xb, gpw_ref[0], preferred_element_type=jnp.float32) + gpb_ref[0]
    u = jnp.dot(xb, upw_ref[0], preferred_element_type=jnp.float32) + upb_ref[0]
    inter = g * (1.0 / (1.0 + jnp.exp(-g))) * u
    out_ref[...] = (
        jnp.dot(inter, dw_ref[0], preferred_element_type=jnp.float32) + db_ref[0])


_ffn_call = pl.pallas_call(
    _ffn_body,
    grid=(E,),
    in_specs=[
        pl.BlockSpec((CAP, H), lambda e: (e, 0)),
        pl.BlockSpec((1, H, F), lambda e: (e, 0, 0)),
        pl.BlockSpec((1, 1, F), lambda e: (e, 0, 0)),
        pl.BlockSpec((1, H, F), lambda e: (e, 0, 0)),
        pl.BlockSpec((1, 1, F), lambda e: (e, 0, 0)),
        pl.BlockSpec((1, F, H), lambda e: (e, 0, 0)),
        pl.BlockSpec((1, 1, H), lambda e: (e, 0, 0)),
    ],
    out_specs=pl.BlockSpec((CAP, H), lambda e: (e, 0)),
    out_shape=jax.ShapeDtypeStruct((BUF_ROWS, H), jnp.float32),
    compiler_params=pltpu.CompilerParams(
        dimension_semantics=("arbitrary",)),
)


def _combine_body(oute_hbm, slot_hbm, wk_hbm, final_hbm, idx_v, rows_v, w_v, sem):
    wid = lax.axis_index("s") * NC + lax.axis_index("c")
    base = wid * TPW
    pltpu.sync_copy(slot_hbm.at[pl.ds(base, TPW)], idx_v)
    pltpu.sync_copy(wk_hbm.at[pl.ds(base, TPW)], w_v)
    pltpu.async_copy(oute_hbm.at[idx_v], rows_v, sem).wait()

    def group_fn(g, carry):
        wreg = w_v[pl.ds(g * L, L)]
        for k in range(L):
            w = wreg[k]
            i = g * L + k

            def chunk_fn(j, c, i=i, w=w):
                for u in range(4):
                    sl = pl.ds((j * 4 + u) * L, L)
                    rows_v[i, sl] = rows_v[i, sl] * w
                return c

            carry = lax.fori_loop(0, H // (4 * L), chunk_fn, carry)
        return carry

    lax.fori_loop(0, TPW // L, group_fn, 0)
    pltpu.sync_copy(rows_v, final_hbm.at[pl.ds(base, TPW)])


def kernel(hidden_states, gate_w, gate_b, up_w, up_b, gp_w, gp_b, down_w, down_b):
    b, s, h = hidden_states.shape
    x = hidden_states.reshape(T, H)
    dispatch, combine = _sc_kernels()
    slotd, slotc, wk = _router_call(x, gate_w, gate_b.reshape(1, E))
    buf = dispatch(x, slotd.reshape(T))
    oute = _ffn_call(
        buf,
        gp_w, gp_b.reshape(E, 1, F),
        up_w, up_b.reshape(E, 1, F),
        down_w, down_b.reshape(E, 1, H),
    )
    final = combine(oute, slotc.reshape(T), wk.reshape(T))
    return final.reshape(b, s, h)


# speedup vs baseline: 1.6429x; 1.0066x over previous
import functools

import jax
import jax.numpy as jnp
from jax import lax
from jax.experimental import pallas as pl
from jax.experimental.pallas import tpu as pltpu
from jax.experimental.pallas import tpu_sc as plsc

E = 64
CAP = 128
H = 1024
F = 512
T = 2048
DUMMY = E * CAP
BUF_ROWS = E * CAP + CAP

NC = 2
NS = 16
NW = NC * NS
L = 16
TPW = T // NW


def _router_body(x_ref, gw_ref, gb_ref, slotd_ref, slotc_ref, wk_ref):
    x = x_ref[...]
    gw = gw_ref[...]
    logits = jnp.dot(x, gw, preferred_element_type=jnp.float32) + gb_ref[...]
    lmax = jnp.max(logits, axis=1, keepdims=True)
    sumexp = jnp.sum(jnp.exp(logits - lmax), axis=1, keepdims=True)
    p = 1.0 / sumexp
    ids = lax.broadcasted_iota(jnp.int32, (T, E), 1)
    eid = jnp.min(jnp.where(logits == lmax, ids, E), axis=1, keepdims=True)
    oh = (ids == eid).astype(jnp.int32)
    cs = oh
    shift = 1
    while shift < T:
        cs = cs + jnp.concatenate(
            [jnp.zeros((shift, E), jnp.int32), cs[: T - shift]], axis=0)
        shift *= 2
    pos = jnp.sum(cs * oh, axis=1, keepdims=True) - 1
    keep = pos < CAP
    slotd_ref[...] = jnp.where(keep, eid * CAP + pos, DUMMY)
    slotc_ref[...] = jnp.where(keep, eid * CAP + pos, DUMMY)
    wk_ref[...] = jnp.where(keep, p, 0.0)


_router_call = pl.pallas_call(
    _router_body,
    out_shape=(
        jax.ShapeDtypeStruct((T, 1), jnp.int32),
        jax.ShapeDtypeStruct((T, 1), jnp.int32),
        jax.ShapeDtypeStruct((T, 1), jnp.float32),
    ),
)


def _dispatch_body(x_hbm, slot_hbm, wk_hbm, buf_hbm, wbuf_hbm,
                   idx_v, rows_v, w_v, wrep_v, sem):
    wid = lax.axis_index("s") * NC + lax.axis_index("c")
    base = wid * TPW
    pltpu.sync_copy(slot_hbm.at[pl.ds(base, TPW)], idx_v)
    pltpu.sync_copy(x_hbm.at[pl.ds(base, TPW)], rows_v)
    pltpu.sync_copy(wk_hbm.at[pl.ds(base, TPW)], w_v)
    copy = pltpu.async_copy(rows_v, buf_hbm.at[idx_v], sem)
    for g in range(TPW // L):
        wreg = w_v[pl.ds(g * L, L)]
        for k in range(L):
            wrep_v[g * L + k, pl.ds(0, L)] = jnp.full((L,), wreg[k], jnp.float32)
    copy.wait()
    pltpu.async_copy(wrep_v, wbuf_hbm.at[idx_v], sem).wait()


@functools.cache
def _sc_kernels():
    mesh = plsc.VectorSubcoreMesh(
        core_axis_name="c", subcore_axis_name="s", num_cores=NC, num_subcores=NS)
    dispatch = pl.kernel(
        _dispatch_body,
        out_type=(
            jax.ShapeDtypeStruct((BUF_ROWS, H), jnp.float32),
            jax.ShapeDtypeStruct((BUF_ROWS, 128), jnp.float32),
        ),
        mesh=mesh,
        scratch_types=[
            pltpu.VMEM((TPW,), jnp.int32),
            pltpu.VMEM((TPW, H), jnp.float32),
            pltpu.VMEM((TPW,), jnp.float32),
            pltpu.VMEM((TPW, 128), jnp.float32),
            pltpu.SemaphoreType.DMA,
        ],
    )
    combine = pl.kernel(
        _combine_body,
        out_type=jax.ShapeDtypeStruct((T, H), jnp.float32),
        mesh=mesh,
        scratch_types=[
            pltpu.VMEM((TPW,), jnp.int32),
            pltpu.VMEM((TPW, H), jnp.float32),
            pltpu.SemaphoreType.DMA,
        ],
    )
    return dispatch, combine


def _ffn_body(xb_ref, wb_ref, gpw_ref, gpb_ref, upw_ref, upb_ref, dw_ref, db_ref,
              out_ref):
    e = pl.program_id(0)

    @pl.when(e < E)
    def _compute():
        xb = xb_ref[...]
        g = jnp.dot(xb, gpw_ref[0], preferred_element_type=jnp.float32) + gpb_ref[0]
        u = jnp.dot(xb, upw_ref[0], preferred_element_type=jnp.float32) + upb_ref[0]
        inter = g * (1.0 / (1.0 + jnp.exp(-g))) * u
        o = jnp.dot(inter, dw_ref[0], preferred_element_type=jnp.float32) + db_ref[0]
        out_ref[...] = o * wb_ref[...][:, 0:1]

    @pl.when(e == E)
    def _zero_drop_bin():
        out_ref[...] = jnp.zeros((CAP, H), jnp.float32)


_ffn_call = pl.pallas_call(
    _ffn_body,
    grid=(E + 1,),
    in_specs=[
        pl.BlockSpec((CAP, H), lambda e: (e, 0)),
        pl.BlockSpec((CAP, 128), lambda e: (e, 0)),
        pl.BlockSpec((1, H, F), lambda e: (jnp.minimum(e, E - 1), 0, 0)),
        pl.BlockSpec((1, 1, F), lambda e: (jnp.minimum(e, E - 1), 0, 0)),
        pl.BlockSpec((1, H, F), lambda e: (jnp.minimum(e, E - 1), 0, 0)),
        pl.BlockSpec((1, 1, F), lambda e: (jnp.minimum(e, E - 1), 0, 0)),
        pl.BlockSpec((1, F, H), lambda e: (jnp.minimum(e, E - 1), 0, 0)),
        pl.BlockSpec((1, 1, H), lambda e: (jnp.minimum(e, E - 1), 0, 0)),
    ],
    out_specs=pl.BlockSpec((CAP, H), lambda e: (e, 0)),
    out_shape=jax.ShapeDtypeStruct((BUF_ROWS, H), jnp.float32),
    compiler_params=pltpu.CompilerParams(
        dimension_semantics=("arbitrary",)),
)


def _combine_body(oute_hbm, slot_hbm, final_hbm, idx_v, rows_v, sem):
    wid = lax.axis_index("s") * NC + lax.axis_index("c")
    base = wid * TPW
    pltpu.sync_copy(slot_hbm.at[pl.ds(base, TPW)], idx_v)
    pltpu.async_copy(oute_hbm.at[idx_v], rows_v, sem).wait()
    pltpu.sync_copy(rows_v, final_hbm.at[pl.ds(base, TPW)])


def kernel(hidden_states, gate_w, gate_b, up_w, up_b, gp_w, gp_b, down_w, down_b):
    b, s, h = hidden_states.shape
    x = hidden_states.reshape(T, H)
    dispatch, combine = _sc_kernels()
    slotd, slotc, wk = _router_call(x, gate_w, gate_b.reshape(1, E))
    buf, wbuf = dispatch(x, slotd.reshape(T), wk.reshape(T))
    oute = _ffn_call(
        buf, wbuf,
        gp_w, gp_b.reshape(E, 1, F),
        up_w, up_b.reshape(E, 1, F),
        down_w, down_b.reshape(E, 1, H),
    )
    final = combine(oute, slotc.reshape(T))
    return final.reshape(b, s, h)
